# baseline (speedup 1.0000x reference)
import jax
import jax.numpy as jnp
from jax import lax
from jax.experimental import pallas as pl
from jax.experimental.pallas import tpu as pltpu


def _dotT(a, b):
    return lax.dot_general(a, b, (((1,), (1,)), ((), ())),
                           preferred_element_type=jnp.float32)


def _routed_kernel(x_ref, gate_w_ref, wg_ref, wu_ref, wd_ref,
                   out_ref, mask_ref):
    e = pl.program_id(0)
    j = pl.program_id(1)
    x = x_ref[...]

    @pl.when((e == 0) & (j == 0))
    def _init():
        scores = _dotT(x, gate_w_ref[...])
        n, n_exp = scores.shape
        idx = lax.broadcasted_iota(jnp.int32, (n, n_exp), 1)
        m1 = jnp.max(scores, axis=1, keepdims=True)
        a1 = jnp.min(jnp.where(scores == m1, idx, n_exp), axis=1, keepdims=True)
        sel1 = idx == a1
        scores2 = jnp.where(sel1, jnp.float32(-jnp.inf), scores)
        m2 = jnp.max(scores2, axis=1, keepdims=True)
        a2 = jnp.min(jnp.where(scores2 == m2, idx, n_exp), axis=1, keepdims=True)
        sel2 = idx == a2
        w1 = jax.nn.sigmoid(m1 - m2)
        mask_ref[...] = (w1 * sel1.astype(jnp.float32)
                         + (1.0 - w1) * sel2.astype(jnp.float32))
        out_ref[...] = jnp.zeros_like(out_ref)

    mask = mask_ref[...]
    col = lax.broadcasted_iota(jnp.int32, mask.shape, 1) == e
    me = jnp.sum(jnp.where(col, mask, 0.0), axis=1, keepdims=True)
    g = _dotT(x, wg_ref[0])
    u = _dotT(x, wu_ref[0])
    h = jax.nn.silu(g) * u * me
    out_ref[...] += _dotT(h, wd_ref[0])


def _shared_kernel(x_ref, sgw_ref, routed_ref, w1_ref, w3_ref, w2_ref,
                   out_ref, sg_ref):
    j = pl.program_id(0)
    x = x_ref[...]

    @pl.when(j == 0)
    def _init():
        sg_ref[...] = jax.nn.sigmoid(_dotT(x, sgw_ref[...]))
        out_ref[...] = routed_ref[...]

    s1 = _dotT(x, w1_ref[...])
    s3 = _dotT(x, w3_ref[...])
    sh = jax.nn.silu(s1) * s3
    out_ref[...] += sg_ref[...] * _dotT(sh, w2_ref[...])


def kernel(x, gate_w, w_gate, w_up, w_down, mlp_w1, mlp_w3, mlp_w2, shared_gate_w):
    B, T, D = x.shape
    E, H, _ = w_gate.shape
    HS = mlp_w1.shape[0]
    N = B * T
    x_flat = x.reshape(N, D)

    n_j = 11
    Hc = H // n_j
    routed = pl.pallas_call(
        _routed_kernel,
        grid=(E, n_j),
        in_specs=[
            pl.BlockSpec((N, D), lambda e, j: (0, 0)),
            pl.BlockSpec((E, D), lambda e, j: (0, 0)),
            pl.BlockSpec((1, Hc, D), lambda e, j: (e, j, 0)),
            pl.BlockSpec((1, Hc, D), lambda e, j: (e, j, 0)),
            pl.BlockSpec((1, D, Hc), lambda e, j: (e, 0, j)),
        ],
        out_specs=pl.BlockSpec((N, D), lambda e, j: (0, 0)),
        out_shape=jax.ShapeDtypeStruct((N, D), jnp.float32),
        scratch_shapes=[pltpu.VMEM((N, E), jnp.float32)],
    )(x_flat, gate_w, w_gate, w_up, w_down)

    n_s = 11
    HSc = HS // n_s
    out = pl.pallas_call(
        _shared_kernel,
        grid=(n_s,),
        in_specs=[
            pl.BlockSpec((N, D), lambda j: (0, 0)),
            pl.BlockSpec((1, D), lambda j: (0, 0)),
            pl.BlockSpec((N, D), lambda j: (0, 0)),
            pl.BlockSpec((HSc, D), lambda j: (j, 0)),
            pl.BlockSpec((HSc, D), lambda j: (j, 0)),
            pl.BlockSpec((D, HSc), lambda j: (0, j)),
        ],
        out_specs=pl.BlockSpec((N, D), lambda j: (0, 0)),
        out_shape=jax.ShapeDtypeStruct((N, D), jnp.float32),
        scratch_shapes=[pltpu.VMEM((N, 1), jnp.float32)],
    )(x_flat, shared_gate_w, routed, mlp_w1, mlp_w3, mlp_w2)
    return out.reshape(B, T, D)


# speedup vs baseline: 1.5747x; 1.5747x over previous
import jax
import jax.numpy as jnp
from jax import lax
from jax.experimental import pallas as pl
from jax.experimental.pallas import tpu as pltpu


def _dotT(a, b):
    return lax.dot_general(a, b, (((1,), (1,)), ((), ())),
                           preferred_element_type=jnp.float32)


def _dotTb(a, b):
    return lax.dot_general(a.astype(jnp.bfloat16), b.astype(jnp.bfloat16),
                           (((1,), (1,)), ((), ())),
                           preferred_element_type=jnp.float32)


def _routed_kernel(x_ref, gate_w_ref, wg_ref, wu_ref, wd_ref,
                   out_ref, mask_ref):
    e = pl.program_id(0)
    j = pl.program_id(1)
    x = x_ref[...]

    @pl.when((e == 0) & (j == 0))
    def _init():
        scores = _dotT(x, gate_w_ref[...])
        n, n_exp = scores.shape
        idx = lax.broadcasted_iota(jnp.int32, (n, n_exp), 1)
        m1 = jnp.max(scores, axis=1, keepdims=True)
        a1 = jnp.min(jnp.where(scores == m1, idx, n_exp), axis=1, keepdims=True)
        sel1 = idx == a1
        scores2 = jnp.where(sel1, jnp.float32(-jnp.inf), scores)
        m2 = jnp.max(scores2, axis=1, keepdims=True)
        a2 = jnp.min(jnp.where(scores2 == m2, idx, n_exp), axis=1, keepdims=True)
        sel2 = idx == a2
        w1 = jax.nn.sigmoid(m1 - m2)
        mask_ref[...] = (w1 * sel1.astype(jnp.float32)
                         + (1.0 - w1) * sel2.astype(jnp.float32))
        out_ref[...] = jnp.zeros_like(out_ref)

    mask = mask_ref[...]
    col = lax.broadcasted_iota(jnp.int32, mask.shape, 1) == e
    me = jnp.sum(jnp.where(col, mask, 0.0), axis=1, keepdims=True)
    g = _dotTb(x, wg_ref[0])
    u = _dotTb(x, wu_ref[0])
    h = jax.nn.silu(g) * u * me
    out_ref[...] += _dotTb(h, wd_ref[0])


def _shared_kernel(x_ref, sgw_ref, routed_ref, w1_ref, w3_ref, w2_ref,
                   out_ref, sg_ref):
    j = pl.program_id(0)
    x = x_ref[...]

    @pl.when(j == 0)
    def _init():
        sg_ref[...] = jax.nn.sigmoid(_dotT(x, sgw_ref[...]))
        out_ref[...] = routed_ref[...]

    s1 = _dotTb(x, w1_ref[...])
    s3 = _dotTb(x, w3_ref[...])
    sh = jax.nn.silu(s1) * s3
    out_ref[...] += sg_ref[...] * _dotTb(sh, w2_ref[...])


def kernel(x, gate_w, w_gate, w_up, w_down, mlp_w1, mlp_w3, mlp_w2, shared_gate_w):
    B, T, D = x.shape
    E, H, _ = w_gate.shape
    HS = mlp_w1.shape[0]
    N = B * T
    x_flat = x.reshape(N, D)

    n_j = 1
    Hc = H // n_j
    routed = pl.pallas_call(
        _routed_kernel,
        grid=(E, n_j),
        in_specs=[
            pl.BlockSpec((N, D), lambda e, j: (0, 0)),
            pl.BlockSpec((E, D), lambda e, j: (0, 0)),
            pl.BlockSpec((1, Hc, D), lambda e, j: (e, j, 0)),
            pl.BlockSpec((1, Hc, D), lambda e, j: (e, j, 0)),
            pl.BlockSpec((1, D, Hc), lambda e, j: (e, 0, j)),
        ],
        out_specs=pl.BlockSpec((N, D), lambda e, j: (0, 0)),
        out_shape=jax.ShapeDtypeStruct((N, D), jnp.float32),
        scratch_shapes=[pltpu.VMEM((N, E), jnp.float32)],
    )(x_flat, gate_w, w_gate, w_up, w_down)

    n_s = 11
    HSc = HS // n_s
    out = pl.pallas_call(
        _shared_kernel,
        grid=(n_s,),
        in_specs=[
            pl.BlockSpec((N, D), lambda j: (0, 0)),
            pl.BlockSpec((1, D), lambda j: (0, 0)),
            pl.BlockSpec((N, D), lambda j: (0, 0)),
            pl.BlockSpec((HSc, D), lambda j: (j, 0)),
            pl.BlockSpec((HSc, D), lambda j: (j, 0)),
            pl.BlockSpec((D, HSc), lambda j: (0, j)),
        ],
        out_specs=pl.BlockSpec((N, D), lambda j: (0, 0)),
        out_shape=jax.ShapeDtypeStruct((N, D), jnp.float32),
        scratch_shapes=[pltpu.VMEM((N, 1), jnp.float32)],
    )(x_flat, shared_gate_w, routed, mlp_w1, mlp_w3, mlp_w2)
    return out.reshape(B, T, D)


# speedup vs baseline: 1.6214x; 1.0297x over previous
import functools

import jax
import jax.numpy as jnp
from jax import lax
from jax.experimental import pallas as pl
from jax.experimental.pallas import tpu as pltpu


def _dotT(a, b):
    return lax.dot_general(a, b, (((1,), (1,)), ((), ())),
                           preferred_element_type=jnp.float32)


def _dotTb(a, b):
    return lax.dot_general(a.astype(jnp.bfloat16), b.astype(jnp.bfloat16),
                           (((1,), (1,)), ((), ())),
                           preferred_element_type=jnp.float32)


def _moe_kernel(x_ref, gate_w_ref, sgw_ref, wg_ref, wu_ref, wd_ref,
                w1_ref, w3_ref, w2_ref, out_ref, mask_ref, sg_ref, *, n_e):
    i = pl.program_id(0)
    x = x_ref[...]

    @pl.when(i == 0)
    def _init():
        scores = _dotT(x, gate_w_ref[...])
        n, n_exp = scores.shape
        idx = lax.broadcasted_iota(jnp.int32, (n, n_exp), 1)
        m1 = jnp.max(scores, axis=1, keepdims=True)
        a1 = jnp.min(jnp.where(scores == m1, idx, n_exp), axis=1, keepdims=True)
        sel1 = idx == a1
        scores2 = jnp.where(sel1, jnp.float32(-jnp.inf), scores)
        m2 = jnp.max(scores2, axis=1, keepdims=True)
        a2 = jnp.min(jnp.where(scores2 == m2, idx, n_exp), axis=1, keepdims=True)
        sel2 = idx == a2
        w1 = jax.nn.sigmoid(m1 - m2)
        mask_ref[...] = (w1 * sel1.astype(jnp.float32)
                         + (1.0 - w1) * sel2.astype(jnp.float32))
        sg_ref[...] = jax.nn.sigmoid(_dotT(x, sgw_ref[...]))
        out_ref[...] = jnp.zeros_like(out_ref)

    @pl.when(i < n_e)
    def _expert():
        mask = mask_ref[...]
        col = lax.broadcasted_iota(jnp.int32, mask.shape, 1) == i
        me = jnp.sum(jnp.where(col, mask, 0.0), axis=1, keepdims=True)
        g = _dotTb(x, wg_ref[0])
        u = _dotTb(x, wu_ref[0])
        h = jax.nn.silu(g) * u * me
        out_ref[...] += _dotTb(h, wd_ref[0])

    @pl.when(i >= n_e)
    def _shared():
        s1 = _dotTb(x, w1_ref[...])
        s3 = _dotTb(x, w3_ref[...])
        sh = jax.nn.silu(s1) * s3
        out_ref[...] += sg_ref[...] * _dotTb(sh, w2_ref[...])


def kernel(x, gate_w, w_gate, w_up, w_down, mlp_w1, mlp_w3, mlp_w2, shared_gate_w):
    B, T, D = x.shape
    E, H, _ = w_gate.shape
    HS = mlp_w1.shape[0]
    N = B * T
    x_flat = x.reshape(N, D)
    n_s = 11
    HSc = HS // n_s
    steps = E + n_s

    def e_idx(i):
        return jnp.minimum(i, E - 1)

    def s_idx(i):
        return jnp.maximum(i - E, 0)

    out = pl.pallas_call(
        functools.partial(_moe_kernel, n_e=E),
        grid=(steps,),
        in_specs=[
            pl.BlockSpec((N, D), lambda i: (0, 0)),
            pl.BlockSpec((E, D), lambda i: (0, 0)),
            pl.BlockSpec((1, D), lambda i: (0, 0)),
            pl.BlockSpec((1, H, D), lambda i: (e_idx(i), 0, 0)),
            pl.BlockSpec((1, H, D), lambda i: (e_idx(i), 0, 0)),
            pl.BlockSpec((1, D, H), lambda i: (e_idx(i), 0, 0)),
            pl.BlockSpec((HSc, D), lambda i: (s_idx(i), 0)),
            pl.BlockSpec((HSc, D), lambda i: (s_idx(i), 0)),
            pl.BlockSpec((D, HSc), lambda i: (0, s_idx(i))),
        ],
        out_specs=pl.BlockSpec((N, D), lambda i: (0, 0)),
        out_shape=jax.ShapeDtypeStruct((N, D), jnp.float32),
        scratch_shapes=[
            pltpu.VMEM((N, E), jnp.float32),
            pltpu.VMEM((N, 1), jnp.float32),
        ],
    )(x_flat, gate_w, shared_gate_w, w_gate, w_up, w_down,
      mlp_w1, mlp_w3, mlp_w2)
    return out.reshape(B, T, D)


# speedup vs baseline: 1.6272x; 1.0036x over previous
import functools

import jax
import jax.numpy as jnp
from jax import lax
from jax.experimental import pallas as pl
from jax.experimental.pallas import tpu as pltpu


def _dotT(a, b):
    return lax.dot_general(a, b, (((1,), (1,)), ((), ())),
                           preferred_element_type=jnp.float32)


def _dotTb(a, b):
    return lax.dot_general(a.astype(jnp.bfloat16), b.astype(jnp.bfloat16),
                           (((1,), (1,)), ((), ())),
                           preferred_element_type=jnp.float32)


def _moe_kernel(x_ref, gate_w_ref, sgw_ref,
                wga_ref, wgb_ref, wua_ref, wub_ref, wda_ref, wdb_ref,
                w1a_ref, w1b_ref, w3a_ref, w3b_ref, w2a_ref, w2b_ref,
                out_ref, mask_ref, sg_ref, *, n_e, dh):
    i = pl.program_id(0)
    x = x_ref[...]
    xa = x[:, :dh]
    xb = x[:, dh:]

    @pl.when(i == 0)
    def _init():
        scores = _dotT(x, gate_w_ref[...])
        n, n_exp = scores.shape
        idx = lax.broadcasted_iota(jnp.int32, (n, n_exp), 1)
        m1 = jnp.max(scores, axis=1, keepdims=True)
        a1 = jnp.min(jnp.where(scores == m1, idx, n_exp), axis=1, keepdims=True)
        sel1 = idx == a1
        scores2 = jnp.where(sel1, jnp.float32(-jnp.inf), scores)
        m2 = jnp.max(scores2, axis=1, keepdims=True)
        a2 = jnp.min(jnp.where(scores2 == m2, idx, n_exp), axis=1, keepdims=True)
        sel2 = idx == a2
        w1 = jax.nn.sigmoid(m1 - m2)
        mask_ref[...] = (w1 * sel1.astype(jnp.float32)
                         + (1.0 - w1) * sel2.astype(jnp.float32))
        sg_ref[...] = jax.nn.sigmoid(_dotT(x, sgw_ref[...]))
        out_ref[...] = jnp.zeros_like(out_ref)

    @pl.when(i < n_e)
    def _expert():
        mask = mask_ref[...]
        col = lax.broadcasted_iota(jnp.int32, mask.shape, 1) == i
        me = jnp.sum(jnp.where(col, mask, 0.0), axis=1, keepdims=True)
        g = _dotTb(xa, wga_ref[0]) + _dotTb(xb, wgb_ref[0])
        u = _dotTb(xa, wua_ref[0]) + _dotTb(xb, wub_ref[0])
        h = jax.nn.silu(g) * u * me
        out_ref[:, :dh] += _dotTb(h, wda_ref[0])
        out_ref[:, dh:] += _dotTb(h, wdb_ref[0])

    @pl.when(i >= n_e)
    def _shared():
        s1 = _dotTb(xa, w1a_ref[...]) + _dotTb(xb, w1b_ref[...])
        s3 = _dotTb(xa, w3a_ref[...]) + _dotTb(xb, w3b_ref[...])
        sh = jax.nn.silu(s1) * s3
        sg = sg_ref[...]
        out_ref[:, :dh] += sg * _dotTb(sh, w2a_ref[...])
        out_ref[:, dh:] += sg * _dotTb(sh, w2b_ref[...])


def kernel(x, gate_w, w_gate, w_up, w_down, mlp_w1, mlp_w3, mlp_w2, shared_gate_w):
    B, T, D = x.shape
    E, H, _ = w_gate.shape
    HS = mlp_w1.shape[0]
    N = B * T
    dh = D // 2
    x_flat = x.reshape(N, D)
    n_s = 11
    HSc = HS // n_s
    steps = E + n_s

    def e_idx(i):
        return jnp.minimum(i, E - 1)

    def s_idx(i):
        return jnp.maximum(i - E, 0)

    out = pl.pallas_call(
        functools.partial(_moe_kernel, n_e=E, dh=dh),
        grid=(steps,),
        in_specs=[
            pl.BlockSpec((N, D), lambda i: (0, 0)),
            pl.BlockSpec((E, D), lambda i: (0, 0)),
            pl.BlockSpec((1, D), lambda i: (0, 0)),
            pl.BlockSpec((1, H, dh), lambda i: (e_idx(i), 0, 0)),
            pl.BlockSpec((1, H, dh), lambda i: (e_idx(i), 0, 1)),
            pl.BlockSpec((1, H, dh), lambda i: (e_idx(i), 0, 0)),
            pl.BlockSpec((1, H, dh), lambda i: (e_idx(i), 0, 1)),
            pl.BlockSpec((1, dh, H), lambda i: (e_idx(i), 0, 0)),
            pl.BlockSpec((1, dh, H), lambda i: (e_idx(i), 1, 0)),
            pl.BlockSpec((HSc, dh), lambda i: (s_idx(i), 0)),
            pl.BlockSpec((HSc, dh), lambda i: (s_idx(i), 1)),
            pl.BlockSpec((HSc, dh), lambda i: (s_idx(i), 0)),
            pl.BlockSpec((HSc, dh), lambda i: (s_idx(i), 1)),
            pl.BlockSpec((dh, HSc), lambda i: (0, s_idx(i))),
            pl.BlockSpec((dh, HSc), lambda i: (1, s_idx(i))),
        ],
        out_specs=pl.BlockSpec((N, D), lambda i: (0, 0)),
        out_shape=jax.ShapeDtypeStruct((N, D), jnp.float32),
        scratch_shapes=[
            pltpu.VMEM((N, E), jnp.float32),
            pltpu.VMEM((N, 1), jnp.float32),
        ],
    )(x_flat, gate_w, shared_gate_w,
      w_gate, w_gate, w_up, w_up, w_down, w_down,
      mlp_w1, mlp_w1, mlp_w3, mlp_w3, mlp_w2, mlp_w2)
    return out.reshape(B, T, D)
